# baseline (speedup 1.0000x reference)
import jax
import jax.numpy as jnp
from jax import lax
from jax.experimental import pallas as pl

N, CH, RES, K, S, V, HID = 8, 96, 38, 6, 3, 512, 48
LH = (RES - K) // S + 1
L = LH * LH
KK = K * K
MID = CH * 3
CIN = CH // K
E = LH * S
PIX = RES * RES


def _fused_kernel(x_ref, wvct_ref, wvc_ref,
                  wim1_ref, bim1_ref, wim2_ref, bim2_ref,
                  wt_ref, bt_ref, wp_ref, bp_ref, wpsi_ref, bpsi_ref,
                  wr1_ref, br1_ref, wr2_ref, br2_ref,
                  w1_ref, b1_ref, wfull_ref, bfull_ref,
                  out_ref):
    f32 = jnp.float32
    xT = x_ref[0].reshape(CH, PIX).T
    wvct = wvct_ref[...]

    inv_pix = lax.rsqrt(jnp.sum(xT * xT, axis=1, keepdims=True))
    s = jnp.dot(xT, wvct, preferred_element_type=f32)
    inv_wn = lax.rsqrt(jnp.sum(wvct * wvct, axis=0, keepdims=True))
    t3 = (s * inv_pix).reshape(RES, RES, V)

    g = jnp.max(t3[:LH * S + S].reshape(LH + 1, S, RES, V), axis=1)
    gr = jnp.maximum(g[:LH], g[1:])
    h = jnp.max(gr[:, :LH * S + S].reshape(LH, LH + 1, S, V), axis=2)
    pm = jnp.maximum(h[:, :LH], h[:, 1:]).reshape(L, V) * inv_wn

    mx = jnp.max(pm, axis=1, keepdims=True)
    iota_v = lax.broadcasted_iota(jnp.int32, (L, V), 1)
    ch = jnp.min(jnp.where(pm >= mx, iota_v, V), axis=1)
    onehot_ch = (iota_v == ch[:, None]).astype(f32)

    xT3 = xT.reshape(RES, RES, CH)
    quads = []
    for u in range(2):
        row = []
        for v in range(2):
            q = xT3[3 * u:3 * u + E, 3 * v:3 * v + E, :]
            q = q.reshape(LH, S, LH, S, CH).transpose(0, 2, 1, 3, 4)
            row.append(q)
        quads.append(jnp.stack(row, axis=3))
    xr6 = jnp.stack(quads, axis=2)
    xr3 = xr6.reshape(L, KK, CH)

    wvc_n = wvc_ref[...] * inv_wn.reshape(V, 1)
    w_sel = jnp.dot(onehot_ch, wvc_n, preferred_element_type=f32)
    inv3 = inv_pix.reshape(RES, RES)
    iq = [jnp.stack([inv3[3 * u:3 * u + E, 3 * v:3 * v + E]
                     .reshape(LH, S, LH, S).transpose(0, 2, 1, 3)
                     for v in range(2)], axis=3) for u in range(2)]
    invr = jnp.stack(iq, axis=2).reshape(L, KK)
    sim_row = jnp.sum(xr3 * w_sel[:, None, :], axis=2) * invr
    mxs = jnp.max(sim_row, axis=1, keepdims=True)
    iota_k = lax.broadcasted_iota(jnp.int32, (L, KK), 1)
    col = jnp.min(jnp.where(sim_row >= mxs, iota_k, KK), axis=1)
    onehot_col = (iota_k == col[:, None]).astype(f32)

    max_act = jnp.sum(xr3 * onehot_col[:, :, None], axis=1)
    vc = jnp.dot(onehot_ch, wvc_ref[...], preferred_element_type=f32)
    integ = max_act * mxs + vc * (1.0 - mxs)

    cat = jnp.concatenate([integ, sim_row], axis=1)
    hh = jnp.dot(cat, wim1_ref[...], preferred_element_type=f32) + bim1_ref[...]
    hh = jnp.where(hh >= 0, hh, 0.2 * hh)
    feat = jnp.tanh(jnp.dot(hh, wim2_ref[...], preferred_element_type=f32)
                    + bim2_ref[...])

    xt = jnp.dot(feat, wt_ref[...], preferred_element_type=f32) + bt_ref[...]
    xph = jnp.dot(feat, wp_ref[...], preferred_element_type=f32) + bp_ref[...]
    xpsi = jnp.dot(feat, wpsi_ref[...], preferred_element_type=f32) + bpsi_ref[...]
    att = jax.nn.softmax(jnp.dot(xph, xt.T, preferred_element_type=f32), axis=1)
    xadd = jnp.dot(att, xpsi, preferred_element_type=f32)
    xadd = jnp.dot(xadd, wr1_ref[...], preferred_element_type=f32) + br1_ref[...]
    xadd = jnp.where(xadd >= 0, xadd, 0.2 * xadd)
    xadd = jnp.tanh(jnp.dot(xadd, wr2_ref[...], preferred_element_type=f32)
                    + br2_ref[...])
    feat = feat + xadd

    z = jnp.dot(feat, w1_ref[...], preferred_element_type=f32) + b1_ref[...]
    z = jnp.maximum(z, 0.0)
    r = jnp.dot(z, wfull_ref[...], preferred_element_type=f32) + bfull_ref[...]

    r6 = r.reshape(LH, LH, 2, S, 2, S, CH)
    acc = jnp.zeros((RES, RES, CH), dtype=f32)
    for u in range(2):
        for v in range(2):
            blk = r6[:, :, u, :, v, :, :]
            blk = blk.transpose(0, 2, 1, 3, 4).reshape(E, E, CH)
            pad = ((3 * u, RES - E - 3 * u), (3 * v, RES - E - 3 * v), (0, 0))
            acc = acc + jnp.pad(blk, pad)
    out_ref[0] = acc.reshape(PIX, CH).T.reshape(CH, RES, RES)


@jax.jit
def kernel(x, Wvc, Wim1, bim1, Wim2, bim2, Wt, bt, Wp, bp, Wpsi, bpsi,
           Wr1, br1, Wr2, br2, W1, b1, W2, b2, Wc):
    w2r = W2.reshape(MID, CIN, KK)
    wfull = jnp.einsum('hck,oc->hko', w2r, Wc).reshape(MID, KK * CH)
    bfull = jnp.einsum('ck,oc->ko', b2.reshape(CIN, KK), Wc).reshape(1, KK * CH)

    row = lambda b: b.reshape(1, -1)
    full = lambda a: pl.BlockSpec(a.shape, lambda i: (0,) * a.ndim)

    args = (x, Wvc.T, Wvc,
            Wim1, row(bim1), Wim2, row(bim2),
            Wt, row(bt), Wp, row(bp), Wpsi, row(bpsi),
            Wr1, row(br1), Wr2, row(br2),
            W1, row(b1), wfull, bfull)
    in_specs = [pl.BlockSpec((1, CH, RES, RES), lambda i: (i, 0, 0, 0))]
    in_specs += [full(a) for a in args[1:]]

    return pl.pallas_call(
        _fused_kernel,
        grid=(N,),
        in_specs=in_specs,
        out_specs=pl.BlockSpec((1, CH, RES, RES), lambda i: (i, 0, 0, 0)),
        out_shape=jax.ShapeDtypeStruct((N, CH, RES, RES), jnp.float32),
    )(*args)


# speedup vs baseline: 1.0011x; 1.0011x over previous
import numpy as np

import jax
import jax.numpy as jnp
from jax import lax
from jax.experimental import pallas as pl

N, CH, RES, K, S, V, HID = 8, 96, 38, 6, 3, 512, 48
LH = (RES - K) // S + 1
L = LH * LH
KK = K * K
MID = CH * 3
CIN = CH // K
E = LH * S
PIX = RES * RES


def _fused_kernel(x_ref, wvct_ref, wvc2_ref,
                  base_ref, delta_ref,
                  wim1_ref, bim1_ref, wim2_ref, bim2_ref,
                  wt_ref, bt_ref, wp_ref, bp_ref, wpsi_ref, bpsi_ref,
                  wr1_ref, br1_ref, wr2_ref, br2_ref,
                  w1_ref, b1_ref, wfull_ref, bfull_ref,
                  out_ref):
    f32 = jnp.float32
    xT = x_ref[0].reshape(CH, PIX).T
    wvct = wvct_ref[...]

    inv_pix = lax.rsqrt(jnp.sum(xT * xT, axis=1, keepdims=True))
    inv_wn = lax.rsqrt(jnp.sum(wvct * wvct, axis=0, keepdims=True))
    s = jnp.dot(xT, wvct, preferred_element_type=f32)
    t3 = (s * inv_pix).reshape(RES, RES, V)

    g = jnp.max(t3[:E + S].reshape(LH + 1, S, RES, V), axis=1)
    gr = jnp.maximum(g[:LH], g[1:])
    h = jnp.max(gr[:, :E + S].reshape(LH, LH + 1, S, V), axis=2)
    pm = jnp.maximum(h[:, :LH], h[:, 1:]).reshape(L, V) * inv_wn

    mx = jnp.max(pm, axis=1, keepdims=True)
    iota_v = lax.broadcasted_iota(jnp.int32, (L, V), 1)
    ch = jnp.min(jnp.where(pm >= mx, iota_v, V), axis=1)
    onehot_ch = (iota_v == ch[:, None]).astype(f32)

    wvc2 = wvc2_ref[...]
    w_seln = jnp.dot(onehot_ch, wvc2[:V], preferred_element_type=f32)
    vc = jnp.dot(onehot_ch, wvc2[V:], preferred_element_type=f32)

    xT3 = xT.reshape(RES, RES, CH)
    quads = []
    for u in range(2):
        qrow = []
        for v in range(2):
            q = xT3[3 * u:3 * u + E, 3 * v:3 * v + E, :]
            q = q.reshape(LH, S, LH, S, CH).transpose(0, 2, 1, 3, 4)
            qrow.append(q)
        quads.append(jnp.stack(qrow, axis=3))
    xr6 = jnp.stack(quads, axis=2)
    xr3 = xr6.reshape(L, KK, CH)
    inv3 = inv_pix.reshape(RES, RES)
    iq = [jnp.stack([inv3[3 * u:3 * u + E, 3 * v:3 * v + E]
                     .reshape(LH, S, LH, S).transpose(0, 2, 1, 3)
                     for v in range(2)], axis=3) for u in range(2)]
    invr = jnp.stack(iq, axis=2).reshape(L, KK)
    sim_row = jnp.sum(xr3 * w_seln[:, None, :], axis=2) * invr

    mxs = jnp.max(sim_row, axis=1, keepdims=True)
    iota_k = lax.broadcasted_iota(jnp.int32, (L, KK), 1)
    col = jnp.min(jnp.where(sim_row >= mxs, iota_k, KK), axis=1,
                  keepdims=True)
    onehot_col = (iota_k == col).astype(f32)
    max_act = jnp.sum(xr3 * onehot_col[:, :, None], axis=1)
    integ = max_act * mxs + vc * (1.0 - mxs)

    cat = jnp.concatenate([integ, sim_row], axis=1)
    hh = jnp.dot(cat, wim1_ref[...], preferred_element_type=f32) + bim1_ref[...]
    hh = jnp.where(hh >= 0, hh, 0.2 * hh)
    feat = jnp.tanh(jnp.dot(hh, wim2_ref[...], preferred_element_type=f32)
                    + bim2_ref[...])

    xt = jnp.dot(feat, wt_ref[...], preferred_element_type=f32) + bt_ref[...]
    xph = jnp.dot(feat, wp_ref[...], preferred_element_type=f32) + bp_ref[...]
    xpsi = jnp.dot(feat, wpsi_ref[...], preferred_element_type=f32) + bpsi_ref[...]
    att = jax.nn.softmax(jnp.dot(xph, xt.T, preferred_element_type=f32), axis=1)
    xadd = jnp.dot(att, xpsi, preferred_element_type=f32)
    xadd = jnp.dot(xadd, wr1_ref[...], preferred_element_type=f32) + br1_ref[...]
    xadd = jnp.where(xadd >= 0, xadd, 0.2 * xadd)
    xadd = jnp.tanh(jnp.dot(xadd, wr2_ref[...], preferred_element_type=f32)
                    + br2_ref[...])
    feat = feat + xadd

    z = jnp.dot(feat, w1_ref[...], preferred_element_type=f32) + b1_ref[...]
    z = jnp.maximum(z, 0.0)
    r = jnp.dot(z, wfull_ref[...], preferred_element_type=f32) + bfull_ref[...]

    r6 = r.reshape(LH, LH, 2, 2, S, S, CH)
    acc = jnp.zeros((RES, RES, CH), dtype=f32)
    for u in range(2):
        for v in range(2):
            blk = r6[:, :, u, v, :, :, :]
            blk = blk.transpose(0, 2, 1, 3, 4).reshape(E, E, CH)
            pad = ((3 * u, RES - E - 3 * u), (3 * v, RES - E - 3 * v), (0, 0))
            acc = acc + jnp.pad(blk, pad)
    out_ref[0] = acc.reshape(PIX, CH).T.reshape(CH, RES, RES)


@jax.jit
def kernel(x, Wvc, Wim1, bim1, Wim2, bim2, Wt, bt, Wp, bp, Wpsi, bpsi,
           Wr1, br1, Wr2, br2, W1, b1, W2, b2, Wc):
    f32 = jnp.float32
    inv_wn = 1.0 / jnp.sqrt(jnp.sum(Wvc * Wvc, axis=1, keepdims=True))
    wvc2 = jnp.concatenate([Wvc * inv_wn, Wvc], axis=0)

    w2r = W2.reshape(MID, CIN, KK)
    wfull = jnp.einsum('hck,oc->hko', w2r, Wc).reshape(MID, K, K, CH)
    wfull = (wfull.reshape(MID, 2, S, 2, S, CH)
             .transpose(0, 1, 3, 2, 4, 5).reshape(MID, KK * CH))
    bfull = jnp.einsum('ck,oc->ko', b2.reshape(CIN, KK), Wc).reshape(K, K, CH)
    bfull = (bfull.reshape(2, S, 2, S, CH)
             .transpose(0, 2, 1, 3, 4).reshape(1, KK * CH))

    ii, jj = np.meshgrid(np.arange(LH), np.arange(LH), indexing='ij')
    base = jnp.asarray((3 * ii * RES + 3 * jj).reshape(L, 1), dtype=f32)
    ki, kj = np.meshgrid(np.arange(K), np.arange(K), indexing='ij')
    delta = jnp.asarray((ki * RES + kj).reshape(1, KK), dtype=f32)

    row = lambda b: b.reshape(1, -1)
    full = lambda a: pl.BlockSpec(a.shape, lambda i: (0,) * a.ndim)

    args = (x, Wvc.T, wvc2, base, delta,
            Wim1, row(bim1), Wim2, row(bim2),
            Wt, row(bt), Wp, row(bp), Wpsi, row(bpsi),
            Wr1, row(br1), Wr2, row(br2),
            W1, row(b1), wfull, bfull)
    in_specs = [pl.BlockSpec((1, CH, RES, RES), lambda i: (i, 0, 0, 0))]
    in_specs += [full(a) for a in args[1:]]

    return pl.pallas_call(
        _fused_kernel,
        grid=(N,),
        in_specs=in_specs,
        out_specs=pl.BlockSpec((1, CH, RES, RES), lambda i: (i, 0, 0, 0)),
        out_shape=jax.ShapeDtypeStruct((N, CH, RES, RES), jnp.float32),
    )(*args)


# speedup vs baseline: 1.0451x; 1.0440x over previous
import jax
import jax.numpy as jnp
from jax import lax
from jax.experimental import pallas as pl

N, CH, RES, K, S, V, HID = 8, 96, 38, 6, 3, 512, 48
LH = (RES - K) // S + 1
L = LH * LH
KK = K * K
MID = CH * 3
CIN = CH // K
E = LH * S


def _fused_kernel(xr_ref, wvct_ref, wvc_ref,
                  wim1_ref, bim1_ref, wim2_ref, bim2_ref,
                  wt_ref, bt_ref, wp_ref, bp_ref, wpsi_ref, bpsi_ref,
                  wr1_ref, br1_ref, wr2_ref, br2_ref,
                  w1_ref, b1_ref, wfull_ref, bfull_ref,
                  out_ref):
    f32 = jnp.float32
    xr = xr_ref[0]
    wvct = wvct_ref[...]

    s = jnp.dot(xr, wvct, preferred_element_type=f32)
    inv_xn = lax.rsqrt(jnp.sum(xr * xr, axis=1, keepdims=True))
    inv_wn = lax.rsqrt(jnp.sum(wvct * wvct, axis=0, keepdims=True))
    t3 = (s * inv_xn).reshape(L, KK, V)

    pm = jnp.max(t3, axis=1) * inv_wn
    mx = jnp.max(pm, axis=1, keepdims=True)
    iota_v = lax.broadcasted_iota(jnp.int32, (L, V), 1)
    ch = jnp.min(jnp.where(pm >= mx, iota_v, V), axis=1)
    onehot_ch = (iota_v == ch[:, None]).astype(f32)

    wvc_n = wvc_ref[...] * inv_wn.reshape(V, 1)
    w_sel = jnp.dot(onehot_ch, wvc_n, preferred_element_type=f32)
    xr3 = xr.reshape(L, KK, CH)
    inv_xn3 = inv_xn.reshape(L, KK, 1)
    sim_row = jnp.sum(xr3 * w_sel[:, None, :], axis=2,
                      keepdims=True) * inv_xn3
    sim_row = sim_row.reshape(L, KK)
    mxs = jnp.max(sim_row, axis=1, keepdims=True)
    iota_k = lax.broadcasted_iota(jnp.int32, (L, KK), 1)
    col = jnp.min(jnp.where(sim_row >= mxs, iota_k, KK), axis=1)
    onehot_col = (iota_k == col[:, None]).astype(f32)

    max_act = jnp.sum(xr3 * onehot_col[:, :, None], axis=1)
    vc = jnp.dot(onehot_ch, wvc_ref[...], preferred_element_type=f32)
    integ = max_act * mxs + vc * (1.0 - mxs)

    cat = jnp.concatenate([integ, sim_row], axis=1)
    h = jnp.dot(cat, wim1_ref[...], preferred_element_type=f32) + bim1_ref[...]
    h = jnp.where(h >= 0, h, 0.2 * h)
    feat = jnp.tanh(jnp.dot(h, wim2_ref[...], preferred_element_type=f32)
                    + bim2_ref[...])

    xt = jnp.dot(feat, wt_ref[...], preferred_element_type=f32) + bt_ref[...]
    xph = jnp.dot(feat, wp_ref[...], preferred_element_type=f32) + bp_ref[...]
    xpsi = jnp.dot(feat, wpsi_ref[...], preferred_element_type=f32) + bpsi_ref[...]
    logits = jnp.dot(xph, xt.T, preferred_element_type=f32)
    att = jax.nn.softmax(logits, axis=1)
    xadd = jnp.dot(att, xpsi, preferred_element_type=f32)
    xadd = jnp.dot(xadd, wr1_ref[...], preferred_element_type=f32) + br1_ref[...]
    xadd = jnp.where(xadd >= 0, xadd, 0.2 * xadd)
    xadd = jnp.tanh(jnp.dot(xadd, wr2_ref[...], preferred_element_type=f32)
                    + br2_ref[...])
    feat = feat + xadd

    z = jnp.dot(feat, w1_ref[...], preferred_element_type=f32) + b1_ref[...]
    z = jnp.maximum(z, 0.0)
    r = jnp.dot(z, wfull_ref[...], preferred_element_type=f32) + bfull_ref[...]

    r6 = r.reshape(LH, LH, 2, S, 2, S, CH)
    out_ref[...] = jnp.zeros_like(out_ref)
    for u in range(2):
        for v in range(2):
            blk = r6[:, :, u, :, v, :, :]
            blk = blk.transpose(0, 2, 1, 3, 4).reshape(E, E, CH)
            out_ref[0, 3 * u:3 * u + E, 3 * v:3 * v + E, :] += blk


@jax.jit
def kernel(x, Wvc, Wim1, bim1, Wim2, bim2, Wt, bt, Wp, bp, Wpsi, bpsi,
           Wr1, br1, Wr2, br2, W1, b1, W2, b2, Wc):
    f32 = jnp.float32
    g = jnp.stack(
        [x[:, :, 3 * u:3 * u + E, 3 * v:3 * v + E].reshape(N, CH, LH, S, LH, S)
         for u in range(2) for v in range(2)],
        axis=0).reshape(2, 2, N, CH, LH, S, LH, S)
    xr = g.transpose(2, 4, 6, 0, 5, 1, 7, 3).reshape(N, L * KK, CH)

    w2r = W2.reshape(MID, CIN, KK)
    wfull = jnp.einsum('hck,oc->hko', w2r, Wc).reshape(MID, KK * CH)
    bfull = jnp.einsum('ck,oc->ko', b2.reshape(CIN, KK), Wc).reshape(1, KK * CH)

    row = lambda b: b.reshape(1, -1)
    full = lambda a: pl.BlockSpec(a.shape, lambda i: (0,) * a.ndim)

    args = (xr, Wvc.T, Wvc,
            Wim1, row(bim1), Wim2, row(bim2),
            Wt, row(bt), Wp, row(bp), Wpsi, row(bpsi),
            Wr1, row(br1), Wr2, row(br2),
            W1, row(b1), wfull, bfull)
    in_specs = [pl.BlockSpec((1, L * KK, CH), lambda i: (i, 0, 0))]
    in_specs += [full(a) for a in args[1:]]

    out = pl.pallas_call(
        _fused_kernel,
        grid=(N,),
        in_specs=in_specs,
        out_specs=pl.BlockSpec((1, RES, RES, CH), lambda i: (i, 0, 0, 0)),
        out_shape=jax.ShapeDtypeStruct((N, RES, RES, CH), f32),
    )(*args)
    return out.transpose(0, 3, 1, 2)
